# Initial kernel scaffold; baseline (speedup 1.0000x reference)
import jax
import jax.numpy as jnp
from jax.experimental import pallas as pl


def kernel(point_cloud, Wp0, Wp1, Wc, Wn, Wl):
    raise NotImplementedError("write your pallas kernel here")


# speedup vs baseline: 6.0134x; 6.0134x over previous
import math

import jax
import jax.numpy as jnp
from jax import lax
from jax.experimental import pallas as pl

_K = 8
_BIG = 1e30
_RB = 256


def _mm(a, b):
    return jnp.dot(a, b, preferred_element_type=jnp.float32,
                   precision=lax.Precision.HIGHEST)


def _mm_nt(a, b):
    return lax.dot_general(a, b, (((1,), (1,)), ((), ())),
                           preferred_element_type=jnp.float32,
                           precision=lax.Precision.HIGHEST)


def _col_iota(nq, ns):
    return lax.broadcasted_iota(jnp.int32, (nq, ns), 1).astype(jnp.float32)


def _row_iota(nq, ns):
    return lax.broadcasted_iota(jnp.int32, (nq, ns), 0).astype(jnp.float32)


def _top8(d, col):
    ns = d.shape[1]
    idxs = []
    for _ in range(_K):
        m = jnp.min(d, axis=1, keepdims=True)
        cand = jnp.where(d == m, col, float(ns))
        ik = jnp.min(cand, axis=1, keepdims=True)
        idxs.append(ik)
        d = jnp.where(col == ik, _BIG, d)
    return idxs


def _max_gather(idxs, col, feat):
    acc = None
    for ik in idxs:
        v = _mm((col == ik).astype(jnp.float32), feat)
        acc = v if acc is None else jnp.maximum(acc, v)
    return acc


def _adjacency(idxs, col):
    adj = None
    for ik in idxs:
        oh = (col == ik).astype(jnp.float32)
        adj = oh if adj is None else adj + oh
    return adj


def _sel_matrix(nq, ns, stride):
    return (_col_iota(nq, ns) == float(stride) * _row_iota(nq, ns)).astype(
        jnp.float32)


def _dist_rows(xq, xf):
    ones = xq * 0.0 + 1.0
    cn2 = _mm_nt(ones, xf * xf)
    xx = lax.dot_general(xq, xf, (((1,), (1,)), ((), ())),
                         preferred_element_type=jnp.float32,
                         precision=lax.Precision.DEFAULT)
    return cn2 - 2.0 * xx


def _res_gcn(pts, adj, wc_ref, wn_ref, n_blocks):
    inv_k = 1.0 / _K
    for i in range(n_blocks):
        h = jnp.maximum(pts, 0.0)
        center = _mm(h, wc_ref[i])
        neigh = inv_k * _mm(adj, _mm(h, wn_ref[i]))
        pts = pts + center + neigh
    return pts


def _k1(xq_ref, xf_ref, wp0_ref, f_ref, idx_ref):
    r = xq_ref.shape[1]
    n = xf_ref.shape[1]
    xq = xq_ref[0]
    xf = xf_ref[0]
    wp0 = wp0_ref[...]

    col = _col_iota(r, n)
    d = _dist_rows(xq, xf)
    projq = _mm(xq, wp0)
    projf = _mm(xf, wp0)

    lane8 = lax.broadcasted_iota(jnp.int32, (r, _K), 1).astype(jnp.float32)
    idx_out = jnp.zeros((r, _K), jnp.float32)
    f = None
    for k in range(_K):
        m = jnp.min(d, axis=1, keepdims=True)
        cand = jnp.where(d == m, col, float(n))
        ik = jnp.min(cand, axis=1, keepdims=True)
        oh = (col == ik).astype(jnp.float32)
        d = jnp.where(col == ik, _BIG, d)
        v = jnp.maximum(_mm(oh, projf) - projq, 0.0)
        f = v if f is None else jnp.maximum(f, v)
        idx_out = idx_out + jnp.where(lane8 == float(k), ik, 0.0)

    f_ref[0] = f
    idx_ref[0] = idx_out


def _k2(ff_ref, idx_ref, wp1_ref, f2_ref):
    r = idx_ref.shape[1]
    n = ff_ref.shape[1]
    ff = ff_ref[0]
    idx = idx_ref[0]
    h = _mm(ff, wp1_ref[...])
    col = _col_iota(r, n)
    f2 = None
    for k in range(_K):
        ik = lax.slice(idx, (0, k), (r, k + 1))
        v = jnp.maximum(_mm((col == ik).astype(jnp.float32), h), 0.0)
        f2 = v if f2 is None else jnp.maximum(f2, v)
    f2_ref[0] = f2


def _pool_gcn(xf, pts_in, wc_ref, wn_ref, n_src, n_new, stride):
    n = xf.shape[0]
    selq = _sel_matrix(n_new, n, stride * 4)
    sels = _sel_matrix(n_src, n, stride)
    xq = _mm(selq, xf)
    xs = _mm(sels, xf)

    col_qs = _col_iota(n_new, n_src)
    dq = _dist_rows(xq, xs)
    idxp = _top8(dq, col_qs)
    pooled = _max_gather(idxp, col_qs, pts_in)

    sel4 = _sel_matrix(n_new, n_src, 4)
    d_new = _mm_nt(dq, sel4)
    col_n = _col_iota(n_new, n_new)
    idxg = _top8(d_new, col_n)
    adj = _adjacency(idxg, col_n)
    return _res_gcn(pooled, adj, wc_ref, wn_ref, 4)


def _k3(xf_ref, f2_ref, wc_ref, wn_ref, pts_ref):
    n = xf_ref.shape[1]
    pts_ref[0] = _pool_gcn(xf_ref[0], f2_ref[0], wc_ref, wn_ref,
                           n_src=n, n_new=n // 4, stride=1)


def _k4(xf_ref, p1_ref, wc_ref, wn_ref, wl_ref, out_ref):
    n = xf_ref.shape[1]
    pts = _pool_gcn(xf_ref[0], p1_ref[0], wc_ref, wn_ref,
                    n_src=n // 4, n_new=n // 16, stride=4)
    out_ref[0] = _mm(jnp.maximum(pts, 0.0), wl_ref[...])


@jax.jit
def kernel(point_cloud, Wp0, Wp1, Wc, Wn, Wl):
    b, n, c = point_cloud.shape
    x_pad = jnp.pad(point_cloud, ((0, 0), (0, 0), (0, 8 - c)))
    wp0_pad = jnp.pad(Wp0, ((0, 8 - Wp0.shape[0]), (0, 0)))
    nb = n // _RB

    f, idx = pl.pallas_call(
        _k1,
        grid=(b, nb),
        in_specs=[
            pl.BlockSpec((1, _RB, 8), lambda i, j: (i, j, 0)),
            pl.BlockSpec((1, n, 8), lambda i, j: (i, 0, 0)),
            pl.BlockSpec((8, 64), lambda i, j: (0, 0)),
        ],
        out_specs=[
            pl.BlockSpec((1, _RB, 64), lambda i, j: (i, j, 0)),
            pl.BlockSpec((1, _RB, _K), lambda i, j: (i, j, 0)),
        ],
        out_shape=[
            jax.ShapeDtypeStruct((b, n, 64), jnp.float32),
            jax.ShapeDtypeStruct((b, n, _K), jnp.float32),
        ],
    )(x_pad, x_pad, wp0_pad)

    f2 = pl.pallas_call(
        _k2,
        grid=(b, nb),
        in_specs=[
            pl.BlockSpec((1, n, 64), lambda i, j: (i, 0, 0)),
            pl.BlockSpec((1, _RB, _K), lambda i, j: (i, j, 0)),
            pl.BlockSpec((64, 64), lambda i, j: (0, 0)),
        ],
        out_specs=pl.BlockSpec((1, _RB, 64), lambda i, j: (i, j, 0)),
        out_shape=jax.ShapeDtypeStruct((b, n, 64), jnp.float32),
    )(f, idx, Wp1)

    p1 = pl.pallas_call(
        _k3,
        grid=(b,),
        in_specs=[
            pl.BlockSpec((1, n, 8), lambda i: (i, 0, 0)),
            pl.BlockSpec((1, n, 64), lambda i: (i, 0, 0)),
            pl.BlockSpec((4, 64, 64), lambda i: (0, 0, 0)),
            pl.BlockSpec((4, 64, 64), lambda i: (0, 0, 0)),
        ],
        out_specs=pl.BlockSpec((1, n // 4, 64), lambda i: (i, 0, 0)),
        out_shape=jax.ShapeDtypeStruct((b, n // 4, 64), jnp.float32),
    )(x_pad, f2, Wc, Wn)

    out = pl.pallas_call(
        _k4,
        grid=(b,),
        in_specs=[
            pl.BlockSpec((1, n, 8), lambda i: (i, 0, 0)),
            pl.BlockSpec((1, n // 4, 64), lambda i: (i, 0, 0)),
            pl.BlockSpec((4, 64, 64), lambda i: (0, 0, 0)),
            pl.BlockSpec((4, 64, 64), lambda i: (0, 0, 0)),
            pl.BlockSpec((64, 1), lambda i: (0, 0)),
        ],
        out_specs=pl.BlockSpec((1, n // 16, 1), lambda i: (i, 0, 0)),
        out_shape=jax.ShapeDtypeStruct((b, n // 16, 1), jnp.float32),
    )(x_pad, p1, Wc, Wn, Wl)
    return out


# speedup vs baseline: 10.4620x; 1.7398x over previous
import math

import jax
import jax.numpy as jnp
from jax import lax
from jax.experimental import pallas as pl

_K = 8
_BIG = 1e30
_RB = 256


def _mm(a, b):
    return jnp.dot(a, b, preferred_element_type=jnp.float32,
                   precision=lax.Precision.HIGHEST)


def _mm_nt(a, b):
    return lax.dot_general(a, b, (((1,), (1,)), ((), ())),
                           preferred_element_type=jnp.float32,
                           precision=lax.Precision.HIGHEST)


def _split16(feat):
    hi = feat.astype(jnp.bfloat16)
    lo = (feat - hi.astype(jnp.float32)).astype(jnp.bfloat16)
    return hi, lo


def _gmm_split(oh_bf, hi, lo):
    return (jnp.dot(oh_bf, hi, preferred_element_type=jnp.float32)
            + jnp.dot(oh_bf, lo, preferred_element_type=jnp.float32))


def _col_iota(nq, ns):
    return lax.broadcasted_iota(jnp.int32, (nq, ns), 1).astype(jnp.float32)


def _row_iota(nq, ns):
    return lax.broadcasted_iota(jnp.int32, (nq, ns), 0).astype(jnp.float32)


def _top8(d, col):
    ns = d.shape[1]
    idxs = []
    for _ in range(_K):
        m = jnp.min(d, axis=1, keepdims=True)
        cand = jnp.where(d == m, col, float(ns))
        ik = jnp.min(cand, axis=1, keepdims=True)
        idxs.append(ik)
        d = jnp.where(col == ik, _BIG, d)
    return idxs


def _max_gather(idxs, col, feat):
    hi, lo = _split16(feat)
    acc = None
    for ik in idxs:
        v = _gmm_split((col == ik).astype(jnp.bfloat16), hi, lo)
        acc = v if acc is None else jnp.maximum(acc, v)
    return acc


def _adjacency(idxs, col):
    adj = None
    for ik in idxs:
        oh = (col == ik).astype(jnp.bfloat16)
        adj = oh if adj is None else adj + oh
    return adj


def _sel_matrix(nq, ns, stride):
    return (_col_iota(nq, ns) == float(stride) * _row_iota(nq, ns)).astype(
        jnp.float32)


def _dist_rows(xq, xf):
    q2 = jnp.sum(xq * xq, axis=1, keepdims=True)
    xft = xf.T
    s2t = jnp.sum(xft * xft, axis=0, keepdims=True)
    xx = lax.dot_general(xq, xf, (((1,), (1,)), ((), ())),
                         preferred_element_type=jnp.float32,
                         precision=lax.Precision.DEFAULT)
    return (q2 - 2.0 * xx) + s2t


def _res_gcn(pts, adj, wc_ref, wn_ref, n_blocks):
    inv_k = 1.0 / _K
    for i in range(n_blocks):
        h = jnp.maximum(pts, 0.0)
        center = _mm(h, wc_ref[i])
        hw_hi, hw_lo = _split16(_mm(h, wn_ref[i]))
        neigh = inv_k * _gmm_split(adj, hw_hi, hw_lo)
        pts = pts + center + neigh
    return pts


def _k1(xq_ref, xf_ref, wp0_ref, f_ref, idx_ref):
    r = xq_ref.shape[1]
    n = xf_ref.shape[1]
    xq = xq_ref[0]
    xf = xf_ref[0]
    wp0 = wp0_ref[...]

    col = _col_iota(r, n)
    d = _dist_rows(xq, xf)
    projq = _mm(xq, wp0)
    projf = _mm(xf, wp0)
    pf_hi, pf_lo = _split16(projf)

    lane8 = lax.broadcasted_iota(jnp.int32, (r, _K), 1).astype(jnp.float32)
    idx_out = jnp.zeros((r, _K), jnp.float32)
    f = None
    for k in range(_K):
        m = jnp.min(d, axis=1, keepdims=True)
        cand = jnp.where(d == m, col, float(n))
        ik = jnp.min(cand, axis=1, keepdims=True)
        oh = (col == ik).astype(jnp.bfloat16)
        d = jnp.where(col == ik, _BIG, d)
        v = jnp.maximum(_gmm_split(oh, pf_hi, pf_lo) - projq, 0.0)
        f = v if f is None else jnp.maximum(f, v)
        idx_out = idx_out + jnp.where(lane8 == float(k), ik, 0.0)

    f_ref[0] = f
    idx_ref[0] = idx_out


def _k2(ff_ref, idx_ref, wp1_ref, f2_ref):
    r = idx_ref.shape[1]
    n = ff_ref.shape[1]
    ff = ff_ref[0]
    idx = idx_ref[0]
    h = _mm(ff, wp1_ref[...])
    h_hi, h_lo = _split16(h)
    col = _col_iota(r, n)
    f2 = None
    for k in range(_K):
        ik = lax.slice(idx, (0, k), (r, k + 1))
        v = jnp.maximum(_gmm_split((col == ik).astype(jnp.bfloat16), h_hi, h_lo), 0.0)
        f2 = v if f2 is None else jnp.maximum(f2, v)
    f2_ref[0] = f2


def _pool_gcn(xf, pts_in, wc_ref, wn_ref, n_src, n_new, stride):
    n = xf.shape[0]
    selq = _sel_matrix(n_new, n, stride * 4)
    sels = _sel_matrix(n_src, n, stride)
    xq = _mm(selq, xf)
    xs = _mm(sels, xf)

    col_qs = _col_iota(n_new, n_src)
    dq = _dist_rows(xq, xs)
    idxp = _top8(dq, col_qs)
    pooled = _max_gather(idxp, col_qs, pts_in)

    sel4 = _sel_matrix(n_new, n_src, 4)
    d_new = _mm_nt(dq, sel4)
    col_n = _col_iota(n_new, n_new)
    idxg = _top8(d_new, col_n)
    adj = _adjacency(idxg, col_n)
    return _res_gcn(pooled, adj, wc_ref, wn_ref, 4)


def _k3(xf_ref, f2_ref, wc_ref, wn_ref, pts_ref):
    n = xf_ref.shape[1]
    pts_ref[0] = _pool_gcn(xf_ref[0], f2_ref[0], wc_ref, wn_ref,
                           n_src=n, n_new=n // 4, stride=1)


def _k4(xf_ref, p1_ref, wc_ref, wn_ref, wl_ref, out_ref):
    n = xf_ref.shape[1]
    pts = _pool_gcn(xf_ref[0], p1_ref[0], wc_ref, wn_ref,
                    n_src=n // 4, n_new=n // 16, stride=4)
    out_ref[0] = _mm(jnp.maximum(pts, 0.0), wl_ref[...])


@jax.jit
def kernel(point_cloud, Wp0, Wp1, Wc, Wn, Wl):
    b, n, c = point_cloud.shape
    x_pad = jnp.pad(point_cloud, ((0, 0), (0, 0), (0, 8 - c)))
    wp0_pad = jnp.pad(Wp0, ((0, 8 - Wp0.shape[0]), (0, 0)))
    nb = n // _RB

    f, idx = pl.pallas_call(
        _k1,
        grid=(b, nb),
        in_specs=[
            pl.BlockSpec((1, _RB, 8), lambda i, j: (i, j, 0)),
            pl.BlockSpec((1, n, 8), lambda i, j: (i, 0, 0)),
            pl.BlockSpec((8, 64), lambda i, j: (0, 0)),
        ],
        out_specs=[
            pl.BlockSpec((1, _RB, 64), lambda i, j: (i, j, 0)),
            pl.BlockSpec((1, _RB, _K), lambda i, j: (i, j, 0)),
        ],
        out_shape=[
            jax.ShapeDtypeStruct((b, n, 64), jnp.float32),
            jax.ShapeDtypeStruct((b, n, _K), jnp.float32),
        ],
    )(x_pad, x_pad, wp0_pad)

    f2 = pl.pallas_call(
        _k2,
        grid=(b, nb),
        in_specs=[
            pl.BlockSpec((1, n, 64), lambda i, j: (i, 0, 0)),
            pl.BlockSpec((1, _RB, _K), lambda i, j: (i, j, 0)),
            pl.BlockSpec((64, 64), lambda i, j: (0, 0)),
        ],
        out_specs=pl.BlockSpec((1, _RB, 64), lambda i, j: (i, j, 0)),
        out_shape=jax.ShapeDtypeStruct((b, n, 64), jnp.float32),
    )(f, idx, Wp1)

    p1 = pl.pallas_call(
        _k3,
        grid=(b,),
        in_specs=[
            pl.BlockSpec((1, n, 8), lambda i: (i, 0, 0)),
            pl.BlockSpec((1, n, 64), lambda i: (i, 0, 0)),
            pl.BlockSpec((4, 64, 64), lambda i: (0, 0, 0)),
            pl.BlockSpec((4, 64, 64), lambda i: (0, 0, 0)),
        ],
        out_specs=pl.BlockSpec((1, n // 4, 64), lambda i: (i, 0, 0)),
        out_shape=jax.ShapeDtypeStruct((b, n // 4, 64), jnp.float32),
    )(x_pad, f2, Wc, Wn)

    out = pl.pallas_call(
        _k4,
        grid=(b,),
        in_specs=[
            pl.BlockSpec((1, n, 8), lambda i: (i, 0, 0)),
            pl.BlockSpec((1, n // 4, 64), lambda i: (i, 0, 0)),
            pl.BlockSpec((4, 64, 64), lambda i: (0, 0, 0)),
            pl.BlockSpec((4, 64, 64), lambda i: (0, 0, 0)),
            pl.BlockSpec((64, 1), lambda i: (0, 0)),
        ],
        out_specs=pl.BlockSpec((1, n // 16, 1), lambda i: (i, 0, 0)),
        out_shape=jax.ShapeDtypeStruct((b, n // 16, 1), jnp.float32),
    )(x_pad, p1, Wc, Wn, Wl)
    return out


# speedup vs baseline: 13.5964x; 1.2996x over previous
import math

import jax
import jax.numpy as jnp
from jax import lax
from jax.experimental import pallas as pl

_K = 8
_BIG = 1e30
_RB = 256


def _mm(a, b):
    return jnp.dot(a, b, preferred_element_type=jnp.float32,
                   precision=lax.Precision.HIGHEST)


def _strided_rows(x, stride):
    if stride == 1:
        return x
    n, c = x.shape
    m = n // stride
    sel = (_col_iota(m, n) ==
           float(stride) * lax.broadcasted_iota(jnp.int32, (m, n), 0)
           .astype(jnp.float32)).astype(jnp.float32)
    return _mm(sel, x)


def _split16(feat):
    hi = feat.astype(jnp.bfloat16)
    lo = (feat - hi.astype(jnp.float32)).astype(jnp.bfloat16)
    return hi, lo


def _gmm_split(oh_bf, hi, lo):
    return (jnp.dot(oh_bf, hi, preferred_element_type=jnp.float32)
            + jnp.dot(oh_bf, lo, preferred_element_type=jnp.float32))


def _col_iota(nq, ns):
    return lax.broadcasted_iota(jnp.int32, (nq, ns), 1).astype(jnp.float32)


def _top8(d, col):
    ns = d.shape[1]
    idxs = []
    for _ in range(_K):
        m = jnp.min(d, axis=1, keepdims=True)
        cand = jnp.where(d == m, col, float(ns))
        ik = jnp.min(cand, axis=1, keepdims=True)
        idxs.append(ik)
        d = jnp.where(col == ik, _BIG, d)
    return idxs


def _max_gather(idxs, col, feat):
    hi, lo = _split16(feat)
    acc = None
    for ik in idxs:
        v = _gmm_split((col == ik).astype(jnp.bfloat16), hi, lo)
        acc = v if acc is None else jnp.maximum(acc, v)
    return acc


def _adjacency(idxs, col):
    adj = None
    for ik in idxs:
        oh = (col == ik).astype(jnp.bfloat16)
        adj = oh if adj is None else adj + oh
    return adj


def _dist_rows(xq, xf):
    q2 = jnp.sum(xq * xq, axis=1, keepdims=True)
    xft = xf.T
    s2t = jnp.sum(xft * xft, axis=0, keepdims=True)
    xx = lax.dot_general(xq, xf, (((1,), (1,)), ((), ())),
                         preferred_element_type=jnp.float32,
                         precision=lax.Precision.DEFAULT)
    return (q2 - 2.0 * xx) + s2t


def _res_gcn(pts, adj, wc_ref, wn_ref, n_blocks):
    inv_k = 1.0 / _K
    for i in range(n_blocks):
        h = jnp.maximum(pts, 0.0)
        center = _mm(h, wc_ref[i])
        hw_hi, hw_lo = _split16(_mm(h, wn_ref[i]))
        neigh = inv_k * _gmm_split(adj, hw_hi, hw_lo)
        pts = pts + center + neigh
    return pts


def _k1(xq_ref, xf_ref, wp0_ref, f_ref, idx_ref):
    r = xq_ref.shape[1]
    n = xf_ref.shape[1]
    xq = xq_ref[0]
    xf = xf_ref[0]
    wp0 = wp0_ref[...]

    col = _col_iota(r, n)
    d = _dist_rows(xq, xf)
    projq = _mm(xq, wp0)
    projf = _mm(xf, wp0)
    pf_hi, pf_lo = _split16(projf)

    lane8 = lax.broadcasted_iota(jnp.int32, (r, _K), 1).astype(jnp.float32)
    idx_out = jnp.zeros((r, _K), jnp.float32)
    g = None
    for k in range(_K):
        m = jnp.min(d, axis=1, keepdims=True)
        cand = jnp.where(d == m, col, float(n))
        ik = jnp.min(cand, axis=1, keepdims=True)
        oh = (col == ik).astype(jnp.bfloat16)
        d = jnp.where(col == ik, _BIG, d)
        v = _gmm_split(oh, pf_hi, pf_lo)
        g = v if g is None else jnp.maximum(g, v)
        idx_out = idx_out + jnp.where(lane8 == float(k), ik, 0.0)

    f_ref[0] = jnp.maximum(g - projq, 0.0)
    idx_ref[0] = idx_out


def _k2(ff_ref, idx_ref, wp1_ref, f2_ref):
    n = ff_ref.shape[1]
    h = _mm(ff_ref[0], wp1_ref[...])
    h_hi, h_lo = _split16(h)
    col = _col_iota(_RB, n)
    for jb in range(n // _RB):
        idx = idx_ref[0, jb * _RB:(jb + 1) * _RB]
        g = None
        for k in range(_K):
            ik = lax.slice(idx, (0, k), (_RB, k + 1))
            v = _gmm_split((col == ik).astype(jnp.bfloat16), h_hi, h_lo)
            g = v if g is None else jnp.maximum(g, v)
        f2_ref[0, jb * _RB:(jb + 1) * _RB] = jnp.maximum(g, 0.0)


def _pool_gcn(xf, pts_in, wc_ref, wn_ref, n_src, n_new, stride):
    xq = _strided_rows(xf, stride * 4)
    xs = _strided_rows(xf, stride)

    col_qs = _col_iota(n_new, n_src)
    dq = _dist_rows(xq, xs)
    idxp = _top8(dq, col_qs)
    pooled = _max_gather(idxp, col_qs, pts_in)

    d_new = _dist_rows(xq, xq)
    col_n = _col_iota(n_new, n_new)
    idxg = _top8(d_new, col_n)
    adj = _adjacency(idxg, col_n)
    return _res_gcn(pooled, adj, wc_ref, wn_ref, 4)


def _k3(xf_ref, f2_ref, wc_ref, wn_ref, pts_ref):
    n = xf_ref.shape[1]
    pts_ref[0] = _pool_gcn(xf_ref[0], f2_ref[0], wc_ref, wn_ref,
                           n_src=n, n_new=n // 4, stride=1)


def _k4(xf_ref, p1_ref, wc_ref, wn_ref, wl_ref, out_ref):
    n = xf_ref.shape[1]
    pts = _pool_gcn(xf_ref[0], p1_ref[0], wc_ref, wn_ref,
                    n_src=n // 4, n_new=n // 16, stride=4)
    out_ref[0] = _mm(jnp.maximum(pts, 0.0), wl_ref[...])


@jax.jit
def kernel(point_cloud, Wp0, Wp1, Wc, Wn, Wl):
    b, n, c = point_cloud.shape
    x_pad = jnp.pad(point_cloud, ((0, 0), (0, 0), (0, 8 - c)))
    wp0_pad = jnp.pad(Wp0, ((0, 8 - Wp0.shape[0]), (0, 0)))
    nb = n // _RB

    f, idx = pl.pallas_call(
        _k1,
        grid=(b, nb),
        in_specs=[
            pl.BlockSpec((1, _RB, 8), lambda i, j: (i, j, 0)),
            pl.BlockSpec((1, n, 8), lambda i, j: (i, 0, 0)),
            pl.BlockSpec((8, 64), lambda i, j: (0, 0)),
        ],
        out_specs=[
            pl.BlockSpec((1, _RB, 64), lambda i, j: (i, j, 0)),
            pl.BlockSpec((1, _RB, _K), lambda i, j: (i, j, 0)),
        ],
        out_shape=[
            jax.ShapeDtypeStruct((b, n, 64), jnp.float32),
            jax.ShapeDtypeStruct((b, n, _K), jnp.float32),
        ],
    )(x_pad, x_pad, wp0_pad)

    f2 = pl.pallas_call(
        _k2,
        grid=(b,),
        in_specs=[
            pl.BlockSpec((1, n, 64), lambda i: (i, 0, 0)),
            pl.BlockSpec((1, n, _K), lambda i: (i, 0, 0)),
            pl.BlockSpec((64, 64), lambda i: (0, 0)),
        ],
        out_specs=pl.BlockSpec((1, n, 64), lambda i: (i, 0, 0)),
        out_shape=jax.ShapeDtypeStruct((b, n, 64), jnp.float32),
    )(f, idx, Wp1)

    p1 = pl.pallas_call(
        _k3,
        grid=(b,),
        in_specs=[
            pl.BlockSpec((1, n, 8), lambda i: (i, 0, 0)),
            pl.BlockSpec((1, n, 64), lambda i: (i, 0, 0)),
            pl.BlockSpec((4, 64, 64), lambda i: (0, 0, 0)),
            pl.BlockSpec((4, 64, 64), lambda i: (0, 0, 0)),
        ],
        out_specs=pl.BlockSpec((1, n // 4, 64), lambda i: (i, 0, 0)),
        out_shape=jax.ShapeDtypeStruct((b, n // 4, 64), jnp.float32),
    )(x_pad, f2, Wc, Wn)

    out = pl.pallas_call(
        _k4,
        grid=(b,),
        in_specs=[
            pl.BlockSpec((1, n, 8), lambda i: (i, 0, 0)),
            pl.BlockSpec((1, n // 4, 64), lambda i: (i, 0, 0)),
            pl.BlockSpec((4, 64, 64), lambda i: (0, 0, 0)),
            pl.BlockSpec((4, 64, 64), lambda i: (0, 0, 0)),
            pl.BlockSpec((64, 1), lambda i: (0, 0)),
        ],
        out_specs=pl.BlockSpec((1, n // 16, 1), lambda i: (i, 0, 0)),
        out_shape=jax.ShapeDtypeStruct((b, n // 16, 1), jnp.float32),
    )(x_pad, p1, Wc, Wn, Wl)
    return out
